# baseline (speedup 1.0000x reference)
import functools

import jax
import jax.numpy as jnp
from jax import lax
from jax.experimental import pallas as pl
from jax.experimental.pallas import tpu as pltpu
from jax.experimental.pallas import tpu_sc as plsc

V = 1000001
D = 64
NB = 16384
NT = 50
NC, NS = 2, 16
NW = NC * NS
BPW = NB // NW
NBB = BPW // 128
HB = 2
CH = HB * 128
NCHUNK = NT * (NBB // HB)
PERIOD = 6
NFULL = 96


def _make_kernel():
    mesh = plsc.VectorSubcoreMesh(core_axis_name="c", subcore_axis_name="s")

    @functools.partial(
        pl.kernel,
        mesh=mesh,
        compiler_params=pltpu.CompilerParams(
            needs_layout_passes=False, use_tc_tiling_on_sc=False
        ),
        out_type=jax.ShapeDtypeStruct((NT, D // 8, NB // 128, 8, 128), jnp.float32),
        scratch_types=[
            pltpu.VMEM((HB, 128), jnp.int32),
            pltpu.VMEM((HB, 128), jnp.int32),
            pltpu.VMEM((HB, 128), jnp.int32),
            pltpu.VMEM((HB, 128, D), jnp.float32),
            pltpu.VMEM((HB, 128, D), jnp.float32),
            pltpu.VMEM((HB, 128, D), jnp.float32),
            pltpu.VMEM((D // 8, HB, 8, 128), jnp.float32),
            pltpu.VMEM((D // 8, HB, 8, 128), jnp.float32),
            pltpu.SemaphoreType.DMA,
            pltpu.SemaphoreType.DMA,
            pltpu.SemaphoreType.DMA,
        ],
    )
    def gather_kernel(
        table_hbm, idx_hbm, out_hbm,
        idx0, idx1, idx2, rowsa, rowsb, rowsc, cbuf0, cbuf1,
        isem, gsem, osem,
    ):
        wid = lax.axis_index("s") * NC + lax.axis_index("c")
        bb0 = wid * NBB
        idxb = (idx0, idx1, idx2)
        rows = (rowsa, rowsb, rowsc)
        cbuf = (cbuf0, cbuf1)
        lane = lax.iota(jnp.int32, 16)

        idx_row_base = wid * NT * NBB

        def fire_idx(c, ip):
            t = c // 2
            h = c % 2
            pltpu.async_copy(
                idx_hbm.at[pl.ds(idx_row_base + t * NBB + h * HB, HB)],
                idxb[ip],
                isem,
            )

        def drain_idx():
            pltpu.make_async_copy(idx_hbm.at[pl.ds(0, HB)], idxb[0], isem).wait()

        def fire_gather(c, ip, rp):
            for k in range(HB):
                pltpu.async_copy(
                    table_hbm.at[idxb[ip].at[k]], rows[rp].at[k], gsem
                )

        def drain_gather(rp):
            for k in range(HB):
                pltpu.make_async_copy(
                    table_hbm.at[idxb[0].at[k]], rows[rp].at[k], gsem
                ).wait()

        def fire_out(c, cp):
            t = c // 2
            h = c % 2
            pltpu.async_copy(
                cbuf[cp], out_hbm.at[t, :, pl.ds(bb0 + h * HB, HB)], osem
            )

        def drain_out(cp):
            pltpu.make_async_copy(
                cbuf[cp], out_hbm.at[0, :, pl.ds(0, HB)], osem
            ).wait()

        kcol = [k * 16 + lane for k in range(D // 16)]
        pcb = [(k * 16 + lane) // 8 for k in range(D // 16)]
        pci = [(k * 16 + lane) % 8 for k in range(D // 16)]

        def transpose_chunk(ip, rp, cp):
            @plsc.parallel_loop(0, HB * 8 * 4, unroll=2)
            def grp(i):
                g = i // 4
                k = i % 4
                bbl = g // 8
                bmg = g % 8
                bm_v = bmg * 16 + lane
                bbl_v = jnp.full((16,), bbl, jnp.int32)
                iv = plsc.load_gather(idxb[ip], [bbl_v, bm_v])
                m = jnp.where(iv == 0, jnp.float32(0.0), jnp.float32(1.0))
                c16 = k * 16
                for j in range(16):
                    col = c16 + j
                    x = plsc.load_gather(
                        rows[rp], [bbl_v, bm_v, jnp.full((16,), col, jnp.int32)]
                    )
                    plsc.store_scatter(
                        cbuf[cp],
                        [
                            jnp.full((16,), col // 8, jnp.int32),
                            bbl_v,
                            jnp.full((16,), col % 8, jnp.int32),
                            bm_v,
                        ],
                        x * m,
                    )

        def step(c, j, *, tail=False):
            ip = j % 3
            rp = j % 3
            cp = j % 2
            ip2 = (j + 2) % 3
            rp2 = (j + 2) % 3
            if tail:
                if c + 2 < NCHUNK:
                    drain_idx()
                    fire_gather(c + 2, ip2, rp2)
                drain_out(cp)
            else:
                drain_idx()
                fire_gather(c + 2, ip2, rp2)
                pl.when(c >= 2)(lambda: drain_out(cp))
            drain_gather(rp)
            transpose_chunk(ip, rp, cp)
            fire_out(c, cp)
            if tail:
                if c + 3 < NCHUNK:
                    fire_idx(c + 3, j % 3)
            else:
                fire_idx(c + 3, j % 3)

        fire_idx(0, 0)
        fire_idx(1, 1)
        fire_idx(2, 2)
        drain_idx()
        fire_gather(0, 0, 0)
        drain_idx()
        fire_gather(1, 1, 1)

        def outer(u, carry):
            c0 = u * PERIOD
            for j in range(PERIOD):
                step(c0 + j, j)
            return carry

        lax.fori_loop(0, NFULL // PERIOD, outer, 0)
        for c in range(NFULL, NCHUNK):
            step(c, c % PERIOD, tail=True)
        drain_out((NCHUNK - 2) % 2)
        drain_out((NCHUNK - 1) % 2)

    return gather_kernel


@functools.lru_cache(maxsize=1)
def _get_kernel():
    return _make_kernel()


def kernel(ivectors, data):
    idx = (
        data.astype(jnp.int32)
        .T.reshape(NT, NW, NBB, 128)
        .transpose(1, 0, 2, 3)
        .reshape(NW * NT * NBB, 128)
    )
    out5 = _get_kernel()(ivectors, idx)
    return out5.transpose(2, 4, 0, 1, 3).reshape(NB, NT, D)


# speedup vs baseline: 1.0011x; 1.0011x over previous
import functools

import jax
import jax.numpy as jnp
from jax import lax
from jax.experimental import pallas as pl
from jax.experimental.pallas import tpu as pltpu
from jax.experimental.pallas import tpu_sc as plsc

V = 1000001
D = 64
NB = 16384
NT = 50
NC, NS = 2, 16
NW = NC * NS
BPW = NB // NW
NBB = BPW // 128
HB = 2
CH = HB * 128
NCHUNK = NT * (NBB // HB)
PERIOD = 6
NFULL = 96


def _make_kernel():
    mesh = plsc.VectorSubcoreMesh(core_axis_name="c", subcore_axis_name="s")

    @functools.partial(
        pl.kernel,
        mesh=mesh,
        compiler_params=pltpu.CompilerParams(
            needs_layout_passes=False, use_tc_tiling_on_sc=False
        ),
        out_type=jax.ShapeDtypeStruct((NT, D // 8, NB // 128, 8, 128), jnp.float32),
        scratch_types=[
            pltpu.VMEM((HB, 128), jnp.int32),
            pltpu.VMEM((HB, 128), jnp.int32),
            pltpu.VMEM((HB, 128), jnp.int32),
            pltpu.VMEM((HB, 128, D), jnp.float32),
            pltpu.VMEM((HB, 128, D), jnp.float32),
            pltpu.VMEM((HB, 128, D), jnp.float32),
            pltpu.VMEM((D // 8, HB, 8, 128), jnp.float32),
            pltpu.VMEM((D // 8, HB, 8, 128), jnp.float32),
            pltpu.SemaphoreType.DMA,
            pltpu.SemaphoreType.DMA,
            pltpu.SemaphoreType.DMA,
        ],
    )
    def gather_kernel(
        table_hbm, idx_hbm, out_hbm,
        idx0, idx1, idx2, rowsa, rowsb, rowsc, cbuf0, cbuf1,
        isem, gsem, osem,
    ):
        wid = lax.axis_index("s") * NC + lax.axis_index("c")
        bb0 = wid * NBB
        idxb = (idx0, idx1, idx2)
        rows = (rowsa, rowsb, rowsc)
        cbuf = (cbuf0, cbuf1)
        lane = lax.iota(jnp.int32, 16)

        idx_row_base = wid * NT * NBB

        def fire_idx(c, ip):
            t = c // 2
            h = c % 2
            pltpu.async_copy(
                idx_hbm.at[pl.ds(idx_row_base + t * NBB + h * HB, HB)],
                idxb[ip],
                isem,
            )

        def drain_idx():
            pltpu.make_async_copy(idx_hbm.at[pl.ds(0, HB)], idxb[0], isem).wait()

        def fire_gather(c, ip, rp):
            for k in range(HB):
                for j in range(2):
                    pltpu.async_copy(
                        table_hbm.at[idxb[ip].at[k, pl.ds(j * 64, 64)]],
                        rows[rp].at[k, pl.ds(j * 64, 64)],
                        gsem,
                    )

        def drain_gather(rp):
            for k in range(HB):
                for j in range(2):
                    pltpu.make_async_copy(
                        table_hbm.at[idxb[0].at[k, pl.ds(j * 64, 64)]],
                        rows[rp].at[k, pl.ds(j * 64, 64)],
                        gsem,
                    ).wait()

        def fire_out(c, cp):
            t = c // 2
            h = c % 2
            pltpu.async_copy(
                cbuf[cp], out_hbm.at[t, :, pl.ds(bb0 + h * HB, HB)], osem
            )

        def drain_out(cp):
            pltpu.make_async_copy(
                cbuf[cp], out_hbm.at[0, :, pl.ds(0, HB)], osem
            ).wait()

        kcol = [k * 16 + lane for k in range(D // 16)]
        pcb = [(k * 16 + lane) // 8 for k in range(D // 16)]
        pci = [(k * 16 + lane) % 8 for k in range(D // 16)]

        def transpose_chunk(ip, rp, cp):
            @plsc.parallel_loop(0, HB * 8 * 4, unroll=2)
            def grp(i):
                g = i // 4
                k = i % 4
                bbl = g // 8
                bmg = g % 8
                bm_v = bmg * 16 + lane
                bbl_v = jnp.full((16,), bbl, jnp.int32)
                iv = plsc.load_gather(idxb[ip], [bbl_v, bm_v])
                m = jnp.where(iv == 0, jnp.float32(0.0), jnp.float32(1.0))
                c16 = k * 16
                for j in range(16):
                    col = c16 + j
                    x = plsc.load_gather(
                        rows[rp], [bbl_v, bm_v, jnp.full((16,), col, jnp.int32)]
                    )
                    plsc.store_scatter(
                        cbuf[cp],
                        [
                            jnp.full((16,), col // 8, jnp.int32),
                            bbl_v,
                            jnp.full((16,), col % 8, jnp.int32),
                            bm_v,
                        ],
                        x * m,
                    )

        def step(c, j, *, tail=False):
            ip = j % 3
            rp = j % 3
            cp = j % 2
            ip2 = (j + 2) % 3
            rp2 = (j + 2) % 3
            if tail:
                if c + 2 < NCHUNK:
                    drain_idx()
                    fire_gather(c + 2, ip2, rp2)
                drain_out(cp)
            else:
                drain_idx()
                fire_gather(c + 2, ip2, rp2)
                pl.when(c >= 2)(lambda: drain_out(cp))
            drain_gather(rp)
            transpose_chunk(ip, rp, cp)
            fire_out(c, cp)
            if tail:
                if c + 3 < NCHUNK:
                    fire_idx(c + 3, j % 3)
            else:
                fire_idx(c + 3, j % 3)

        fire_idx(0, 0)
        fire_idx(1, 1)
        fire_idx(2, 2)
        drain_idx()
        fire_gather(0, 0, 0)
        drain_idx()
        fire_gather(1, 1, 1)

        def outer(u, carry):
            c0 = u * PERIOD
            for j in range(PERIOD):
                step(c0 + j, j)
            return carry

        lax.fori_loop(0, NFULL // PERIOD, outer, 0)
        for c in range(NFULL, NCHUNK):
            step(c, c % PERIOD, tail=True)
        drain_out((NCHUNK - 2) % 2)
        drain_out((NCHUNK - 1) % 2)

    return gather_kernel


@functools.lru_cache(maxsize=1)
def _get_kernel():
    return _make_kernel()


def kernel(ivectors, data):
    idx = (
        data.astype(jnp.int32)
        .T.reshape(NT, NW, NBB, 128)
        .transpose(1, 0, 2, 3)
        .reshape(NW * NT * NBB, 128)
    )
    out5 = _get_kernel()(ivectors, idx)
    return out5.transpose(2, 4, 0, 1, 3).reshape(NB, NT, D)


# speedup vs baseline: 1.1360x; 1.1347x over previous
import functools

import jax
import jax.numpy as jnp
from jax import lax
from jax.experimental import pallas as pl
from jax.experimental.pallas import tpu as pltpu
from jax.experimental.pallas import tpu_sc as plsc

V = 1000001
D = 64
B = 16384 * 50
NC, NS = 2, 16
NW = NC * NS
IR = 128
G = 5
CH = G * IR
ROWS_PER_W = B // (NW * IR)
NCHUNK = ROWS_PER_W // G


def _idx_splat16(idx_v, flat):
    row = jnp.full((16,), flat // IR, jnp.int32)
    col = jnp.full((16,), flat % IR, jnp.int32)
    return plsc.load_gather(idx_v, [row, col])


def _detect_zero(idx_v, flat_base):
    lane = lax.iota(jnp.int32, 16)
    mn = jnp.full((16,), 1, jnp.int32)
    for t in range(CH // 16):
        flat = flat_base + t * 16
        row = jnp.full((16,), flat // IR, jnp.int32)
        col = jnp.full((16,), flat % IR, jnp.int32) + lane
        mn = jnp.minimum(mn, plsc.load_gather(idx_v, [row, col]))
    nzero = plsc.all_reduce_population_count(mn == 0)
    return nzero[0] > 0


def _fix_zero_rows(idx_v, rows_v, flat_base):
    lane = lax.iota(jnp.int32, 16)

    def fixrow(r, carry):
        iv = _idx_splat16(idx_v, flat_base + r)
        m = jnp.where(iv == 0, jnp.float32(0.0), jnp.float32(1.0))
        rr = jnp.full((16,), r, jnp.int32)
        for k in range(D // 16):
            col = lane + k * 16
            x = plsc.load_gather(rows_v, [rr, col])
            plsc.store_scatter(rows_v, [rr, col], x * m)
        return carry

    lax.fori_loop(0, CH, fixrow, 0)


def _make_kernel():
    mesh = plsc.VectorSubcoreMesh(core_axis_name="c", subcore_axis_name="s")

    @functools.partial(
        pl.kernel,
        mesh=mesh,
        compiler_params=pltpu.CompilerParams(
            needs_layout_passes=False, use_tc_tiling_on_sc=False
        ),
        out_type=jax.ShapeDtypeStruct((B, D), jnp.float32),
        scratch_types=[
            pltpu.VMEM((ROWS_PER_W, IR), jnp.int32),
            pltpu.VMEM((CH, D), jnp.float32),
            pltpu.VMEM((CH, D), jnp.float32),
            pltpu.SemaphoreType.DMA,
            pltpu.SemaphoreType.DMA,
            pltpu.SemaphoreType.DMA,
            pltpu.SemaphoreType.DMA,
        ],
    )
    def gather_kernel(
        table_hbm, idx_hbm, out_hbm,
        idx_v, rows0, rows1, gsem0, gsem1, osem0, osem1,
    ):
        wid = lax.axis_index("s") * NC + lax.axis_index("c")
        row_base = wid * ROWS_PER_W
        idx_base = row_base * IR
        rows = (rows0, rows1)
        gsem = (gsem0, gsem1)
        osem = (osem0, osem1)

        pltpu.sync_copy(idx_hbm.at[pl.ds(row_base, ROWS_PER_W)], idx_v)

        def fire_gather(g, b):
            for j in range(G):
                pltpu.async_copy(
                    table_hbm.at[idx_v.at[g * G + j]],
                    rows[b].at[pl.ds(j * IR, IR)],
                    gsem[b],
                )

        def drain_gather(b):
            pltpu.make_async_copy(
                table_hbm.at[pl.ds(0, CH)], rows[b], gsem[b]
            ).wait()

        def fire_out(g, b):
            pltpu.async_copy(
                rows[b], out_hbm.at[pl.ds(idx_base + g * CH, CH)], osem[b]
            )

        def drain_out(b):
            pltpu.make_async_copy(
                rows[b], out_hbm.at[pl.ds(0, CH)], osem[b]
            ).wait()

        fire_gather(0, 0)

        def outer(k, carry):
            for b in range(2):
                g = k * 2 + b
                nb = 1 - b
                pl.when((g >= 1) & (g + 1 < NCHUNK))(lambda: drain_out(nb))
                pl.when(g + 1 < NCHUNK)(lambda: fire_gather(g + 1, nb))
                has_zero = _detect_zero(idx_v, g * CH)
                drain_gather(b)
                pl.when(has_zero)(
                    lambda: _fix_zero_rows(idx_v, rows[b], g * CH)
                )
                fire_out(g, b)
            return carry

        lax.fori_loop(0, NCHUNK // 2, outer, 0)
        drain_out(0)
        drain_out(1)

    return gather_kernel


@functools.lru_cache(maxsize=1)
def _get_kernel():
    return _make_kernel()


def kernel(ivectors, data):
    idx = data.astype(jnp.int32).reshape(B // IR, IR)
    out = _get_kernel()(ivectors, idx)
    return out.reshape(data.shape[0], data.shape[1], D)
